# baseline (speedup 1.0000x reference)
import functools

import jax
import jax.numpy as jnp
from jax import lax
from jax.experimental import pallas as pl
from jax.experimental.pallas import tpu as pltpu
from jax.experimental.pallas import tpu_sc as plsc

_NC = 2
_NS = 16
_NW = _NC * _NS
_IDXG = 128
_CHUNK = 512


@functools.lru_cache(maxsize=None)
def _make_sc_gather(v, d, n):
    gpc = _CHUNK // _IDXG
    rows_per_w = n // _NW
    nchunk = rows_per_w // _CHUNK
    assert rows_per_w * _NW == n and nchunk * _CHUNK == rows_per_w

    mesh = plsc.VectorSubcoreMesh(core_axis_name="c", subcore_axis_name="s")

    @functools.partial(
        pl.kernel,
        out_type=jax.ShapeDtypeStruct((n, d), jnp.float32),
        mesh=mesh,
        scratch_types=[
            pltpu.VMEM((gpc, _IDXG), jnp.int32),
            pltpu.VMEM((_CHUNK, d), jnp.float32),
            pltpu.SemaphoreType.DMA,
        ],
        compiler_params=pltpu.CompilerParams(use_tc_tiling_on_sc=False),
    )
    def gather(table_hbm, idx_hbm, out_hbm, idx_v, rows_v, sem):
        wid = lax.axis_index("s") * _NC + lax.axis_index("c")

        def chunk(i, carry):
            g0 = (wid * nchunk + i) * gpc
            pltpu.sync_copy(idx_hbm.at[pl.ds(g0, gpc)], idx_v)
            cps = [
                pltpu.async_copy(
                    table_hbm.at[idx_v.at[j]],
                    rows_v.at[pl.ds(j * _IDXG, _IDXG)],
                    sem,
                )
                for j in range(gpc)
            ]
            for c in cps:
                c.wait()
            pltpu.sync_copy(rows_v, out_hbm.at[pl.ds(g0 * _IDXG, _CHUNK)])
            return carry

        lax.fori_loop(0, nchunk, chunk, 0)

    return gather


def _dense_body(e_ref, wpe_ref, wt_ref, b_ref, g_ref, beta_ref, out_ref):
    bt, s, d = e_ref.shape
    x = e_ref[...].reshape(bt * s, d)
    h = jnp.dot(x, wt_ref[...], preferred_element_type=jnp.float32)
    pwt = jnp.dot(wpe_ref[...], wt_ref[...], preferred_element_type=jnp.float32)
    h = h.reshape(bt, s, d) + pwt[None] + b_ref[...][None]
    mean = jnp.mean(h, axis=-1, keepdims=True)
    var = jnp.mean(jnp.square(h - mean), axis=-1, keepdims=True)
    h = (h - mean) * lax.rsqrt(var + 1e-5)
    h = h * g_ref[...][None] + beta_ref[...][None]
    out_ref[...] = h * jax.nn.sigmoid(h)


def kernel(input_ids, wte, wpe, W, b, gamma, beta):
    B, S = input_ids.shape
    V, D = wte.shape
    N = B * S

    idx = input_ids.reshape(N).astype(jnp.int32).reshape(N // _IDXG, _IDXG)
    e = _make_sc_gather(V, D, N)(wte, idx)

    BT = 32
    out = pl.pallas_call(
        _dense_body,
        grid=(B // BT,),
        in_specs=[
            pl.BlockSpec((BT, S, D), lambda i: (i, 0, 0)),
            pl.BlockSpec((S, D), lambda i: (0, 0)),
            pl.BlockSpec((D, D), lambda i: (0, 0)),
            pl.BlockSpec((1, D), lambda i: (0, 0)),
            pl.BlockSpec((1, D), lambda i: (0, 0)),
            pl.BlockSpec((1, D), lambda i: (0, 0)),
        ],
        out_specs=pl.BlockSpec((BT, S, D), lambda i: (i, 0, 0)),
        out_shape=jax.ShapeDtypeStruct((B, S, D), jnp.float32),
    )(
        e.reshape(B, S, D),
        wpe[:S],
        W.T,
        b.reshape(1, D),
        gamma.reshape(1, D),
        beta.reshape(1, D),
    )
    return out


# speedup vs baseline: 1.2181x; 1.2181x over previous
import functools

import jax
import jax.numpy as jnp
from jax import lax
from jax.experimental import pallas as pl
from jax.experimental.pallas import tpu as pltpu
from jax.experimental.pallas import tpu_sc as plsc

_NC = 2
_NS = 16
_NW = _NC * _NS
_IDXG = 128
_CHUNK = 512


@functools.lru_cache(maxsize=None)
def _make_sc_gather(v, d, n):
    gpc = _CHUNK // _IDXG
    rows_per_w = n // _NW
    nchunk = rows_per_w // _CHUNK
    assert rows_per_w * _NW == n and nchunk * _CHUNK == rows_per_w

    mesh = plsc.VectorSubcoreMesh(core_axis_name="c", subcore_axis_name="s")

    @functools.partial(
        pl.kernel,
        out_type=jax.ShapeDtypeStruct((n, d), jnp.float32),
        mesh=mesh,
        scratch_types=[
            pltpu.VMEM((gpc, _IDXG), jnp.int32),
            pltpu.VMEM((_CHUNK, d), jnp.float32),
            pltpu.SemaphoreType.DMA,
        ],
        compiler_params=pltpu.CompilerParams(use_tc_tiling_on_sc=False),
    )
    def gather(table_hbm, idx_hbm, out_hbm, idx_v, rows_v, sem):
        wid = lax.axis_index("s") * _NC + lax.axis_index("c")

        def chunk(i, carry):
            g0 = (wid * nchunk + i) * gpc
            pltpu.sync_copy(idx_hbm.at[pl.ds(g0, gpc)], idx_v)
            cps = [
                pltpu.async_copy(
                    table_hbm.at[idx_v.at[j]],
                    rows_v.at[pl.ds(j * _IDXG, _IDXG)],
                    sem,
                )
                for j in range(gpc)
            ]
            for c in cps:
                c.wait()
            pltpu.sync_copy(rows_v, out_hbm.at[pl.ds(g0 * _IDXG, _CHUNK)])
            return carry

        lax.fori_loop(0, nchunk, chunk, 0)

    return gather


def _dense_body(x_ref, wpe2_ref, w2_ref, b2_ref, g2_ref, beta2_ref, out_ref):
    bt2 = x_ref.shape[0]
    sp = wpe2_ref.shape[0]
    d2 = x_ref.shape[1]
    dm = d2 // 2
    r = lax.broadcasted_iota(jnp.int32, (d2, d2), 0) // dm
    c = lax.broadcasted_iota(jnp.int32, (d2, d2), 1) // dm
    bm = jnp.where(r == c, jnp.float32(1.0 / dm), jnp.float32(0.0))

    w2 = w2_ref[...]
    h = jnp.dot(x_ref[...], w2, preferred_element_type=jnp.float32)
    pwt = jnp.dot(wpe2_ref[...], w2, preferred_element_type=jnp.float32)
    pw_full = jnp.concatenate([pwt] * (bt2 // sp), axis=0)
    h = h + pw_full + b2_ref[...]
    m = jnp.dot(h, bm, preferred_element_type=jnp.float32)
    hc = h - m
    v = jnp.dot(hc * hc, bm, preferred_element_type=jnp.float32)
    hn = hc * lax.rsqrt(v + 1e-5)
    hn = hn * g2_ref[...] + beta2_ref[...]
    out_ref[...] = hn * jax.nn.sigmoid(hn)


def kernel(input_ids, wte, wpe, W, b, gamma, beta):
    B, S = input_ids.shape
    V, D = wte.shape
    N = B * S
    N2 = N // 2
    D2 = 2 * D

    idx = input_ids.reshape(N).astype(jnp.int32).reshape(N // _IDXG, _IDXG)
    e = _make_sc_gather(V, D, N)(wte, idx)

    x2 = e.reshape(N2, D2)
    wt = W.T
    z = jnp.zeros((D, D), jnp.float32)
    w2 = jnp.block([[wt, z], [z, wt]])
    b2 = jnp.concatenate([b, b]).reshape(1, D2)
    g2 = jnp.concatenate([gamma, gamma]).reshape(1, D2)
    beta2 = jnp.concatenate([beta, beta]).reshape(1, D2)
    wpe2 = wpe[:S].reshape(S // 2, D2)

    BT2 = 6400
    out = pl.pallas_call(
        _dense_body,
        grid=(N2 // BT2,),
        in_specs=[
            pl.BlockSpec((BT2, D2), lambda i: (i, 0)),
            pl.BlockSpec((S // 2, D2), lambda i: (0, 0)),
            pl.BlockSpec((D2, D2), lambda i: (0, 0)),
            pl.BlockSpec((1, D2), lambda i: (0, 0)),
            pl.BlockSpec((1, D2), lambda i: (0, 0)),
            pl.BlockSpec((1, D2), lambda i: (0, 0)),
        ],
        out_specs=pl.BlockSpec((BT2, D2), lambda i: (i, 0)),
        out_shape=jax.ShapeDtypeStruct((N2, D2), jnp.float32),
    )(x2, wpe2, w2, b2, g2, beta2)
    return out.reshape(B, S, D)


# speedup vs baseline: 1.2202x; 1.0018x over previous
import functools

import jax
import jax.numpy as jnp
from jax import lax
from jax.experimental import pallas as pl
from jax.experimental.pallas import tpu as pltpu
from jax.experimental.pallas import tpu_sc as plsc

_NC = 2
_NS = 16
_NW = _NC * _NS
_IDXG = 128
_CHUNK = 512


@functools.lru_cache(maxsize=None)
def _make_sc_gather(v, d, n):
    gpc = _CHUNK // _IDXG
    rows_per_w = n // _NW
    nchunk = rows_per_w // _CHUNK
    assert rows_per_w * _NW == n and nchunk * _CHUNK == rows_per_w

    mesh = plsc.VectorSubcoreMesh(core_axis_name="c", subcore_axis_name="s")

    @functools.partial(
        pl.kernel,
        out_type=jax.ShapeDtypeStruct((n // 2, 2 * d), jnp.float32),
        mesh=mesh,
        scratch_types=[
            pltpu.VMEM((gpc, _IDXG), jnp.int32),
            pltpu.VMEM((_CHUNK, d), jnp.float32),
            pltpu.SemaphoreType.DMA,
        ],
        compiler_params=pltpu.CompilerParams(use_tc_tiling_on_sc=False),
    )
    def gather(table_hbm, idx_hbm, out_hbm, idx_v, rows_v, sem):
        wid = lax.axis_index("s") * _NC + lax.axis_index("c")

        def chunk(i, carry):
            g0 = (wid * nchunk + i) * gpc
            pltpu.sync_copy(idx_hbm.at[pl.ds(g0, gpc)], idx_v)
            cps = [
                pltpu.async_copy(
                    table_hbm.at[idx_v.at[j]],
                    rows_v.at[pl.ds(j * _IDXG, _IDXG)],
                    sem,
                )
                for j in range(gpc)
            ]
            for c in cps:
                c.wait()
            r0 = g0 * _IDXG // 2
            pltpu.sync_copy(
                rows_v.at[pl.ds(0, _CHUNK // 2)],
                out_hbm.at[pl.ds(r0, _CHUNK // 2), pl.ds(0, d)],
            )
            pltpu.sync_copy(
                rows_v.at[pl.ds(_CHUNK // 2, _CHUNK // 2)],
                out_hbm.at[pl.ds(r0, _CHUNK // 2), pl.ds(d, d)],
            )
            return carry

        lax.fori_loop(0, nchunk, chunk, 0)

    return gather


def _dense_body(x_ref, wpe2_ref, w2_ref, b2_ref, g2_ref, beta2_ref, out_ref):
    bt2 = x_ref.shape[0]
    sp = wpe2_ref.shape[0]
    d2 = x_ref.shape[1]
    dm = d2 // 2
    r = lax.broadcasted_iota(jnp.int32, (d2, d2), 0) // dm
    c = lax.broadcasted_iota(jnp.int32, (d2, d2), 1) // dm
    bm = jnp.where(r == c, jnp.float32(1.0 / dm), jnp.float32(0.0))

    w2 = w2_ref[...]
    h = jnp.dot(x_ref[...], w2, preferred_element_type=jnp.float32)
    pwt = jnp.dot(wpe2_ref[...], w2, preferred_element_type=jnp.float32)
    pw_full = jnp.concatenate([pwt] * (bt2 // sp), axis=0)
    h = h + pw_full + b2_ref[...]
    m = jnp.dot(h, bm, preferred_element_type=jnp.float32)
    hc = h - m
    v = jnp.dot(hc * hc, bm, preferred_element_type=jnp.float32)
    hn = hc * lax.rsqrt(v + 1e-5)
    hn = hn * g2_ref[...] + beta2_ref[...]
    out_ref[...] = hn * jax.nn.sigmoid(hn)


def kernel(input_ids, wte, wpe, W, b, gamma, beta):
    B, S = input_ids.shape
    V, D = wte.shape
    N = B * S
    N2 = N // 2
    D2 = 2 * D

    ids_flat = input_ids.reshape(N).astype(jnp.int32)
    half = _CHUNK // 2
    idx = (
        jnp.stack(
            [ids_flat[0::2].reshape(-1, half), ids_flat[1::2].reshape(-1, half)],
            axis=1,
        )
        .reshape(N // _IDXG, _IDXG)
    )
    x2 = _make_sc_gather(V, D, N)(wte, idx)
    wt = W.T
    z = jnp.zeros((D, D), jnp.float32)
    w2 = jnp.block([[wt, z], [z, wt]])
    b2 = jnp.concatenate([b, b]).reshape(1, D2)
    g2 = jnp.concatenate([gamma, gamma]).reshape(1, D2)
    beta2 = jnp.concatenate([beta, beta]).reshape(1, D2)
    wpe2 = wpe[:S].reshape(S // 2, D2)

    BT2 = 6400
    out = pl.pallas_call(
        _dense_body,
        grid=(N2 // BT2,),
        in_specs=[
            pl.BlockSpec((BT2, D2), lambda i: (i, 0)),
            pl.BlockSpec((S // 2, D2), lambda i: (0, 0)),
            pl.BlockSpec((D2, D2), lambda i: (0, 0)),
            pl.BlockSpec((1, D2), lambda i: (0, 0)),
            pl.BlockSpec((1, D2), lambda i: (0, 0)),
            pl.BlockSpec((1, D2), lambda i: (0, 0)),
        ],
        out_specs=pl.BlockSpec((BT2, D2), lambda i: (i, 0)),
        out_shape=jax.ShapeDtypeStruct((N2, D2), jnp.float32),
    )(x2, wpe2, w2, b2, g2, beta2)
    return out.reshape(B, S, D)


# speedup vs baseline: 1.7826x; 1.4609x over previous
import functools

import jax
import jax.numpy as jnp
from jax import lax
from jax.experimental import pallas as pl
from jax.experimental.pallas import tpu as pltpu
from jax.experimental.pallas import tpu_sc as plsc

_NC = 2
_NS = 16
_NW = _NC * _NS
_IDXG = 128
_CHUNK = 512


@functools.lru_cache(maxsize=None)
def _make_sc_gather(v, d, n):
    gpc = _CHUNK // _IDXG
    rows_per_w = n // _NW
    nchunk = rows_per_w // _CHUNK
    assert rows_per_w * _NW == n and nchunk * _CHUNK == rows_per_w

    mesh = plsc.VectorSubcoreMesh(core_axis_name="c", subcore_axis_name="s")

    @functools.partial(
        pl.kernel,
        out_type=jax.ShapeDtypeStruct((n // 2, 2 * d), jnp.float32),
        mesh=mesh,
        scratch_types=[
            pltpu.VMEM((gpc, _IDXG), jnp.int32),
            pltpu.VMEM((_CHUNK, d), jnp.float32),
            pltpu.SemaphoreType.DMA,
        ],
        compiler_params=pltpu.CompilerParams(use_tc_tiling_on_sc=False),
    )
    def gather(table_hbm, idx_hbm, out_hbm, idx_v, rows_v, sem):
        wid = lax.axis_index("s") * _NC + lax.axis_index("c")

        def chunk(i, carry):
            g0 = (wid * nchunk + i) * gpc
            pltpu.sync_copy(idx_hbm.at[pl.ds(g0, gpc)], idx_v)
            cps = [
                pltpu.async_copy(
                    table_hbm.at[idx_v.at[j]],
                    rows_v.at[pl.ds(j * _IDXG, _IDXG)],
                    sem,
                )
                for j in range(gpc)
            ]
            for c in cps:
                c.wait()
            r0 = g0 * _IDXG // 2
            pltpu.sync_copy(
                rows_v.at[pl.ds(0, _CHUNK // 2)],
                out_hbm.at[pl.ds(r0, _CHUNK // 2), pl.ds(0, d)],
            )
            pltpu.sync_copy(
                rows_v.at[pl.ds(_CHUNK // 2, _CHUNK // 2)],
                out_hbm.at[pl.ds(r0, _CHUNK // 2), pl.ds(d, d)],
            )
            return carry

        lax.fori_loop(0, nchunk, chunk, 0)

    return gather


def _dense_t_body(x_ref, wpec_ref, w2t_ref, b_ref, g_ref, beta_ref, out_ref):
    bl, d2 = x_ref.shape
    dm = d2 // 2
    x = x_ref[...]
    w2t = w2t_ref[...]
    h_t = lax.dot_general(
        w2t, x, (((1,), (1,)), ((), ())),
        preferred_element_type=jnp.float32,
    )
    pqall = jnp.dot(w2t, wpec_ref[...], preferred_element_type=jnp.float32)
    sel = (
        lax.broadcasted_iota(jnp.int32, pqall.shape, 1) == pl.program_id(0)
    ).astype(jnp.float32)
    pqcol = jnp.sum(pqall * sel, axis=1, keepdims=True)
    h_t = h_t + pqcol + b_ref[...]
    h3 = h_t.reshape(2, dm, bl)
    m = jnp.mean(h3, axis=1, keepdims=True)
    hc = h3 - m
    v = jnp.mean(hc * hc, axis=1, keepdims=True)
    hn = (hc * lax.rsqrt(v + 1e-5)).reshape(d2, bl)
    hn = hn * g_ref[...] + beta_ref[...]
    out_ref[...] = (hn * jax.nn.sigmoid(hn)).reshape(2, dm, bl)


def kernel(input_ids, wte, wpe, W, b, gamma, beta):
    B, S = input_ids.shape
    V, D = wte.shape
    N = B * S
    N2 = N // 2
    D2 = 2 * D

    ids_t = input_ids.T.astype(jnp.int32)
    h1 = ids_t[0::2].reshape(N2)
    h2 = ids_t[1::2].reshape(N2)
    half = _CHUNK // 2
    idx = (
        jnp.stack([h1.reshape(-1, half), h2.reshape(-1, half)], axis=1)
        .reshape(N // _IDXG, _IDXG)
    )
    x2 = _make_sc_gather(V, D, N)(wte, idx)

    z = jnp.zeros((D, D), jnp.float32)
    w2t = jnp.block([[W, z], [z, W]])
    bcol = jnp.concatenate([b, b]).reshape(D2, 1)
    gcol = jnp.concatenate([gamma, gamma]).reshape(D2, 1)
    betacol = jnp.concatenate([beta, beta]).reshape(D2, 1)
    wpec = wpe[:S].reshape(S // 2, D2).T

    out_t = pl.pallas_call(
        _dense_t_body,
        grid=(S // 2,),
        in_specs=[
            pl.BlockSpec((B, D2), lambda i: (i, 0)),
            pl.BlockSpec((D2, S // 2), lambda i: (0, 0)),
            pl.BlockSpec((D2, D2), lambda i: (0, 0)),
            pl.BlockSpec((D2, 1), lambda i: (0, 0)),
            pl.BlockSpec((D2, 1), lambda i: (0, 0)),
            pl.BlockSpec((D2, 1), lambda i: (0, 0)),
        ],
        out_specs=pl.BlockSpec((2, D, B), lambda i: (i, 0, 0)),
        out_shape=jax.ShapeDtypeStruct((S, D, B), jnp.float32),
    )(x2, wpec, w2t, bcol, gcol, betacol)
    return jnp.transpose(out_t, (2, 0, 1))


# speedup vs baseline: 1.7864x; 1.0021x over previous
import functools

import jax
import jax.numpy as jnp
from jax import lax
from jax.experimental import pallas as pl
from jax.experimental.pallas import tpu as pltpu
from jax.experimental.pallas import tpu_sc as plsc

_NC = 2
_NS = 16
_NW = _NC * _NS
_IDXG = 128
_CHUNK = 512


@functools.lru_cache(maxsize=None)
def _make_sc_gather(v, d, n):
    gpc = _CHUNK // _IDXG
    rows_per_w = n // _NW
    nchunk = rows_per_w // _CHUNK
    assert rows_per_w * _NW == n and nchunk * _CHUNK == rows_per_w

    mesh = plsc.VectorSubcoreMesh(core_axis_name="c", subcore_axis_name="s")

    @functools.partial(
        pl.kernel,
        out_type=jax.ShapeDtypeStruct((n // 2, 2 * d), jnp.float32),
        mesh=mesh,
        scratch_types=[
            pltpu.VMEM((gpc, _IDXG), jnp.int32),
            pltpu.VMEM((_CHUNK, d), jnp.float32),
            pltpu.SemaphoreType.DMA,
        ],
        compiler_params=pltpu.CompilerParams(use_tc_tiling_on_sc=False),
    )
    def gather(table_hbm, idx_hbm, out_hbm, idx_v, rows_v, sem):
        wid = lax.axis_index("s") * _NC + lax.axis_index("c")

        def chunk(i, carry):
            g0 = (wid * nchunk + i) * gpc
            pltpu.sync_copy(idx_hbm.at[pl.ds(g0, gpc)], idx_v)
            cps = [
                pltpu.async_copy(
                    table_hbm.at[idx_v.at[j]],
                    rows_v.at[pl.ds(j * _IDXG, _IDXG)],
                    sem,
                )
                for j in range(gpc)
            ]
            for c in cps:
                c.wait()
            r0 = g0 * _IDXG // 2
            pltpu.sync_copy(
                rows_v.at[pl.ds(0, _CHUNK // 2)],
                out_hbm.at[pl.ds(r0, _CHUNK // 2), pl.ds(0, d)],
            )
            pltpu.sync_copy(
                rows_v.at[pl.ds(_CHUNK // 2, _CHUNK // 2)],
                out_hbm.at[pl.ds(r0, _CHUNK // 2), pl.ds(d, d)],
            )
            return carry

        lax.fori_loop(0, nchunk, chunk, 0)

    return gather


def _dense_t_body(x_ref, wpec_ref, w2t_ref, b_ref, g_ref, beta_ref, out_ref):
    bl, d2 = x_ref.shape
    dm = d2 // 2
    x = x_ref[...]
    w2t = w2t_ref[...]
    h_t = lax.dot_general(
        w2t, x, (((1,), (1,)), ((), ())),
        preferred_element_type=jnp.float32,
    )
    pqall = jnp.dot(w2t, wpec_ref[...], preferred_element_type=jnp.float32)
    sel = (
        lax.broadcasted_iota(jnp.int32, pqall.shape, 1) == pl.program_id(0)
    ).astype(jnp.float32)
    pqcol = jnp.sum(pqall * sel, axis=1, keepdims=True)
    h_t = h_t + pqcol + b_ref[...]
    h3 = h_t.reshape(2, dm, bl)
    m = jnp.mean(h3, axis=1, keepdims=True)
    hc = h3 - m
    v = jnp.mean(hc * hc, axis=1, keepdims=True)
    hn = (hc * lax.rsqrt(v + 1e-5)).reshape(d2, bl)
    hn = hn * g_ref[...] + beta_ref[...]
    out_ref[...] = (hn * jax.nn.sigmoid(hn)).reshape(2, dm, bl)


def kernel(input_ids, wte, wpe, W, b, gamma, beta):
    B, S = input_ids.shape
    V, D = wte.shape
    N = B * S
    N2 = N // 2
    D2 = 2 * D

    ids_t = input_ids.T.astype(jnp.int32)
    h1 = ids_t[0::2].reshape(N2)
    h2 = ids_t[1::2].reshape(N2)
    half = _CHUNK // 2
    idx = (
        jnp.stack([h1.reshape(-1, half), h2.reshape(-1, half)], axis=1)
        .reshape(N // _IDXG, _IDXG)
    )
    wte_pk = jax.lax.optimization_barrier(wte.reshape(V // 2, 2 * D))
    wte_lin = wte_pk.reshape(V, D)
    x2 = _make_sc_gather(V, D, N)(wte_lin, idx)

    z = jnp.zeros((D, D), jnp.float32)
    w2t = jnp.block([[W, z], [z, W]])
    bcol = jnp.concatenate([b, b]).reshape(D2, 1)
    gcol = jnp.concatenate([gamma, gamma]).reshape(D2, 1)
    betacol = jnp.concatenate([beta, beta]).reshape(D2, 1)
    wpec = wpe[:S].reshape(S // 2, D2).T

    out_t = pl.pallas_call(
        _dense_t_body,
        grid=(S // 2,),
        in_specs=[
            pl.BlockSpec((B, D2), lambda i: (i, 0)),
            pl.BlockSpec((D2, S // 2), lambda i: (0, 0)),
            pl.BlockSpec((D2, D2), lambda i: (0, 0)),
            pl.BlockSpec((D2, 1), lambda i: (0, 0)),
            pl.BlockSpec((D2, 1), lambda i: (0, 0)),
            pl.BlockSpec((D2, 1), lambda i: (0, 0)),
        ],
        out_specs=pl.BlockSpec((2, D, B), lambda i: (i, 0, 0)),
        out_shape=jax.ShapeDtypeStruct((S, D, B), jnp.float32),
    )(x2, wpec, w2t, bcol, gcol, betacol)
    return jnp.transpose(out_t, (2, 0, 1))
